# baseline (speedup 1.0000x reference)
import dataclasses
import functools

import jax
import jax.numpy as jnp
from jax import lax
from jax.experimental import pallas as pl
from jax.experimental.pallas import tpu as pltpu
from jax.experimental.pallas import tpu_sc as plsc

N = 10000
E = 320000
D_IN = 128
D_H = 128
D_OUT = 64
W = 128

NC = 2
NS = 16
NW = NC * NS
EW = E // NW
SPAN = 624
ZCH = 104
REM = N - NS * SPAN
REM0 = NS * SPAN
VL = 16


def _sc_segment_sum(count_degrees):
    mesh = plsc.VectorSubcoreMesh(
        core_axis_name="c", subcore_axis_name="s",
        num_cores=NC, num_subcores=NS)
    cp = pltpu.CompilerParams()
    if "needs_layout_passes" in pltpu.CompilerParams.__dataclass_fields__:
        cp = dataclasses.replace(cp, needs_layout_passes=False)

    if count_degrees:
        G, NB, AHEAD, CH = 64, 4, 3, 12
    else:
        G, NB, AHEAD, CH = 128, 3, 1, 3
    CHW = CH * G
    NFULL = EW // G
    TAIL = EW - NFULL * G
    NCHUNK = NFULL // CH
    UNROLL = (2 * CH) * NB // __import__("math").gcd(2 * CH, NB)
    TE = min(CH * (NCHUNK - 1) + 2, NFULL - AHEAD)
    T0 = TE - UNROLL * ((TE - (NB - AHEAD)) // UNROLL)

    p_t = jax.ShapeDtypeStruct((NC, N, W), jnp.float32)
    deg_t = jax.ShapeDtypeStruct((NW * N,), jnp.float32)

    scratch = [
        pltpu.VMEM_SHARED((N, W), jnp.float32),
        [pltpu.VMEM((CHW,), jnp.int32)] * 2,
        [pltpu.VMEM((CHW,), jnp.int32)] * 2,
        [pltpu.VMEM((G, W), jnp.float32)] * NB,
        [pltpu.SemaphoreType.DMA] * NB,
        [pltpu.SemaphoreType.DMA] * NB,
        pltpu.SemaphoreType.DMA,
    ]

    if count_degrees:
        scratch.insert(4, pltpu.VMEM((N,), jnp.float32))

        @functools.partial(pl.kernel, compiler_params=cp,
                           out_type=[p_t, deg_t], mesh=mesh,
                           scratch_types=scratch)
        def k(table, src, dst, zhbm, out, deg_out, acc, scnk, dcnk, rows,
              mydeg, gsem, ssem, zsem):
            body(table, src, dst, zhbm, out, deg_out, acc, scnk, dcnk,
                 rows, mydeg, gsem, ssem, zsem)
    else:
        @functools.partial(pl.kernel, compiler_params=cp,
                           out_type=p_t, mesh=mesh,
                           scratch_types=scratch)
        def k(table, src, dst, zhbm, out, acc, scnk, dcnk, rows,
              gsem, ssem, zsem):
            body(table, src, dst, zhbm, out, None, acc, scnk, dcnk,
                 rows, None, gsem, ssem, zsem)

    def body(table, src, dst, zhbm, out, deg_out, acc, scnk, dcnk, rows,
             mydeg, gsem, ssem, zsem):
        c = lax.axis_index("c")
        s = lax.axis_index("s")
        wid = s * NC + c
        base_e = wid * EW

        def count_deg(cb, co):
            if not count_degrees:
                return
            for j in range(G // VL):
                dv = dcnk[cb][pl.ds(co * G + j * VL, VL)]
                cnt, last = plsc.scan_count(dv)
                plsc.addupdate_scatter(
                    mydeg, [dv], cnt.astype(jnp.float32), mask=last)

        def load_chunk(buf, e0):
            pltpu.sync_copy(src.at[pl.ds(e0, CHW)], scnk[buf])
            pltpu.sync_copy(dst.at[pl.ds(e0, CHW)], dcnk[buf])

        def issue_gather(slot, qb, qo):
            pltpu.async_copy(table.at[scnk[qb].at[pl.ds(qo * G, G)]],
                             rows[slot], gsem[slot])

        def wait_gather(slot):
            pltpu.make_async_copy(table.at[scnk[0].at[pl.ds(0, G)]],
                                  rows[slot], gsem[slot]).wait()

        def issue_scatter(slot, cb, co):
            pltpu.async_copy(rows[slot],
                             acc.at[dcnk[cb].at[pl.ds(co * G, G)]],
                             ssem[slot], add=True)

        def wait_scatter(slot):
            pltpu.make_async_copy(rows[slot],
                                  acc.at[dcnk[0].at[pl.ds(0, G)]],
                                  ssem[slot]).wait()

        def step(slot, cb, co, qb, qo, issue, wait_prev):
            if issue:
                slot2 = (slot + AHEAD) % NB
                if wait_prev:
                    wait_scatter(slot2)
                issue_gather(slot2, qb, qo)
            wait_gather(slot)
            issue_scatter(slot, cb, co)
            count_deg(cb, co)

        def emit(g):
            if g % CH == 2 and g // CH + 1 < NCHUNK:
                load_chunk((g // CH + 1) % 2,
                           base_e + (g // CH + 1) * CHW)
            q = g + AHEAD
            step(g % NB, (g // CH) % 2, g % CH, (q // CH) % 2, q % CH,
                 q < NFULL, q >= NB)

        load_chunk(0, base_e)
        for q in range(AHEAD):
            issue_gather(q % NB, 0, q)

        zplan = [(s * SPAN + j * ZCH, ZCH) for j in range(SPAN // ZCH)]
        for r0, nr in zplan:
            pltpu.async_copy(zhbm, acc.at[pl.ds(r0, nr)], zsem)

        @pl.when(s == 0)
        def _ztail():
            pltpu.async_copy(zhbm.at[pl.ds(0, REM)],
                             acc.at[pl.ds(REM0, REM)], zsem)

        if count_degrees:
            @pl.loop(0, N // VL)
            def _zd(i):
                mydeg[pl.ds(i * VL, VL)] = jnp.zeros((VL,), jnp.float32)

        for r0, nr in zplan:
            pltpu.make_async_copy(zhbm, acc.at[pl.ds(r0, nr)], zsem).wait()

        @pl.when(s == 0)
        def _ztailw():
            pltpu.make_async_copy(zhbm.at[pl.ds(0, REM)],
                                  acc.at[pl.ds(REM0, REM)], zsem).wait()

        plsc.subcore_barrier()

        for g in range(T0):
            emit(g)

        @pl.loop(T0, TE, step=UNROLL)
        def _edges(t):
            for db in range(UNROLL):
                gg = T0 + db
                if gg % CH == 2:
                    load_chunk((gg // CH + 1) % 2,
                               base_e + ((t + db) // CH + 1) * CHW)
                q = gg + AHEAD
                step(gg % NB, (gg // CH) % 2, gg % CH,
                     (q // CH) % 2, q % CH, True, True)

        for g in range(TE, NFULL):
            emit(g)
        for q in range(NFULL - NB, NFULL):
            wait_scatter(q % NB)

        e0 = base_e + NFULL * G
        tsl = pl.ds(0, TAIL)
        pltpu.sync_copy(src.at[pl.ds(e0, TAIL)], scnk[0].at[tsl])
        pltpu.sync_copy(dst.at[pl.ds(e0, TAIL)], dcnk[0].at[tsl])
        pltpu.sync_copy(table.at[scnk[0].at[tsl]], rows[0].at[tsl])
        pltpu.sync_copy(rows[0].at[tsl], acc.at[dcnk[0].at[tsl]], add=True)
        if count_degrees:
            for j in range(TAIL // VL):
                dv = dcnk[0][pl.ds(j * VL, VL)]
                cnt, last = plsc.scan_count(dv)
                plsc.addupdate_scatter(
                    mydeg, [dv], cnt.astype(jnp.float32), mask=last)

        plsc.subcore_barrier()
        pltpu.sync_copy(acc.at[pl.ds(s * SPAN, SPAN)],
                        out.at[c, pl.ds(s * SPAN, SPAN)])

        @pl.when(s == 0)
        def _otail():
            pltpu.sync_copy(acc.at[pl.ds(REM0, REM)],
                            out.at[c, pl.ds(REM0, REM)])

        if count_degrees:
            pltpu.sync_copy(mydeg, deg_out.at[pl.ds(wid * N, N)])

    return k


def _dot(a, b):
    return jnp.dot(a, b, precision=lax.Precision.HIGHEST,
                   preferred_element_type=jnp.float32)


BN = 2000


def _mean(p_ref, dp_ref, d):
    agg = p_ref[0] + p_ref[1]
    deg = jnp.maximum(jnp.sum(dp_ref[...], axis=1, keepdims=True), 1.0)
    return agg[:, :d] / deg


def _tc_mid_body(x_ref, p0_ref, dp0_ref, ws0_ref, wn0_ref, b0_ref,
                 ws1_ref, wn1_ref, b1_ref, t1_ref, s1_ref):
    mean = _mean(p0_ref, dp0_ref, D_IN)
    h1 = jax.nn.relu(
        _dot(x_ref[...], ws0_ref[...]) + _dot(mean, wn0_ref[...])
        + b0_ref[...])
    col = lax.broadcasted_iota(jnp.int32, (BN, D_OUT), 1)
    flags = jnp.where(col == 0, 1.0, 0.0).astype(jnp.float32)
    t1_ref[...] = jnp.concatenate([_dot(h1, wn1_ref[...]), flags], axis=1)
    s1_ref[...] = _dot(h1, ws1_ref[...]) + b1_ref[...]


def _tc_out_body(p1_ref, s1_ref, o_ref):
    p = p1_ref[0] + p1_ref[1]
    deg = jnp.maximum(p[:, D_OUT:D_OUT + 1], 1.0)
    o_ref[...] = s1_ref[...] + p[:, :D_OUT] / deg


def kernel(x, edge_index0, edge_index1, W_self0, W_neigh0, b0,
           W_self1, W_neigh1, b1):
    src0, dst0 = edge_index0[0], edge_index0[1]
    src1, dst1 = edge_index1[0], edge_index1[1]
    b0r = b0.reshape(1, D_H)
    b1r = b1.reshape(1, D_OUT)

    zs = jnp.zeros((ZCH, W), jnp.float32)
    p0, degf0 = _sc_segment_sum(True)(x, src0, dst0, zs)
    dp0 = degf0.reshape(NW, N).T

    t1, s1 = pl.pallas_call(
        _tc_mid_body,
        grid=(N // BN,),
        in_specs=[
            pl.BlockSpec((BN, D_IN), lambda i: (i, 0)),
            pl.BlockSpec((NC, BN, W), lambda i: (0, i, 0)),
            pl.BlockSpec((BN, NW), lambda i: (i, 0)),
            pl.BlockSpec((D_IN, D_H), lambda i: (0, 0)),
            pl.BlockSpec((D_IN, D_H), lambda i: (0, 0)),
            pl.BlockSpec((1, D_H), lambda i: (0, 0)),
            pl.BlockSpec((D_H, D_OUT), lambda i: (0, 0)),
            pl.BlockSpec((D_H, D_OUT), lambda i: (0, 0)),
            pl.BlockSpec((1, D_OUT), lambda i: (0, 0)),
        ],
        out_specs=[pl.BlockSpec((BN, W), lambda i: (i, 0)),
                   pl.BlockSpec((BN, D_OUT), lambda i: (i, 0))],
        out_shape=[jax.ShapeDtypeStruct((N, W), jnp.float32),
                   jax.ShapeDtypeStruct((N, D_OUT), jnp.float32)],
    )(x, p0, dp0, W_self0, W_neigh0, b0r, W_self1, W_neigh1, b1r)

    p1 = _sc_segment_sum(False)(t1, src1, dst1, zs)

    out = pl.pallas_call(
        _tc_out_body,
        grid=(N // BN,),
        in_specs=[
            pl.BlockSpec((NC, BN, W), lambda i: (0, i, 0)),
            pl.BlockSpec((BN, D_OUT), lambda i: (i, 0)),
        ],
        out_specs=pl.BlockSpec((BN, D_OUT), lambda i: (i, 0)),
        out_shape=jax.ShapeDtypeStruct((N, D_OUT), jnp.float32),
    )(p1, s1)

    return out


# speedup vs baseline: 1.0803x; 1.0803x over previous
import dataclasses
import functools

import jax
import jax.numpy as jnp
from jax import lax
from jax.experimental import pallas as pl
from jax.experimental.pallas import tpu as pltpu
from jax.experimental.pallas import tpu_sc as plsc

N = 10000
E = 320000
D_IN = 128
D_H = 128
D_OUT = 64
W = 128

NC = 2
NS = 16
NW = NC * NS
EW = E // NW
SPAN = 624
ZCH = 104
REM = N - NS * SPAN
REM0 = NS * SPAN
VL = 16


def _sc_segment_sum(count_degrees):
    mesh = plsc.VectorSubcoreMesh(
        core_axis_name="c", subcore_axis_name="s",
        num_cores=NC, num_subcores=NS)
    cp = pltpu.CompilerParams()
    if "needs_layout_passes" in pltpu.CompilerParams.__dataclass_fields__:
        cp = dataclasses.replace(cp, needs_layout_passes=False)

    G, NB, AHEAD, CH = 64, 4, 3, 12
    CHW = CH * G
    NFULL = EW // G
    TAIL = EW - NFULL * G
    NCHUNK = NFULL // CH
    UNROLL = (2 * CH) * NB // __import__("math").gcd(2 * CH, NB)
    TE = min(CH * (NCHUNK - 1) + 2, NFULL - AHEAD)
    T0 = TE - UNROLL * ((TE - (NB - AHEAD)) // UNROLL)

    p_t = jax.ShapeDtypeStruct((NC, N, W), jnp.float32)
    deg_t = jax.ShapeDtypeStruct((NW * N,), jnp.float32)

    scratch = [
        pltpu.VMEM_SHARED((N, W), jnp.float32),
        [pltpu.VMEM((CHW,), jnp.int32)] * 2,
        [pltpu.VMEM((CHW,), jnp.int32)] * 2,
        [pltpu.VMEM((G, W), jnp.float32)] * NB,
        [pltpu.SemaphoreType.DMA] * NB,
        [pltpu.SemaphoreType.DMA] * NB,
        pltpu.SemaphoreType.DMA,
    ]

    if count_degrees:
        scratch.insert(4, pltpu.VMEM((N,), jnp.float32))

        @functools.partial(pl.kernel, compiler_params=cp,
                           out_type=[p_t, deg_t], mesh=mesh,
                           scratch_types=scratch)
        def k(table, src, dst, zhbm, out, deg_out, acc, scnk, dcnk, rows,
              mydeg, gsem, ssem, zsem):
            body(table, src, dst, zhbm, out, deg_out, acc, scnk, dcnk,
                 rows, mydeg, gsem, ssem, zsem)
    else:
        @functools.partial(pl.kernel, compiler_params=cp,
                           out_type=p_t, mesh=mesh,
                           scratch_types=scratch)
        def k(table, src, dst, zhbm, out, acc, scnk, dcnk, rows,
              gsem, ssem, zsem):
            body(table, src, dst, zhbm, out, None, acc, scnk, dcnk,
                 rows, None, gsem, ssem, zsem)

    def body(table, src, dst, zhbm, out, deg_out, acc, scnk, dcnk, rows,
             mydeg, gsem, ssem, zsem):
        c = lax.axis_index("c")
        s = lax.axis_index("s")
        wid = s * NC + c
        base_e = wid * EW

        def count_deg(cb, co):
            if not count_degrees:
                return
            for j in range(G // VL):
                dv = dcnk[cb][pl.ds(co * G + j * VL, VL)]
                cnt, last = plsc.scan_count(dv)
                plsc.addupdate_scatter(
                    mydeg, [dv], cnt.astype(jnp.float32), mask=last)

        def load_chunk(buf, e0):
            pltpu.sync_copy(src.at[pl.ds(e0, CHW)], scnk[buf])
            pltpu.sync_copy(dst.at[pl.ds(e0, CHW)], dcnk[buf])

        def issue_gather(slot, qb, qo):
            pltpu.async_copy(table.at[scnk[qb].at[pl.ds(qo * G, G)]],
                             rows[slot], gsem[slot])

        def wait_gather(slot):
            pltpu.make_async_copy(table.at[scnk[0].at[pl.ds(0, G)]],
                                  rows[slot], gsem[slot]).wait()

        def issue_scatter(slot, cb, co):
            pltpu.async_copy(rows[slot],
                             acc.at[dcnk[cb].at[pl.ds(co * G, G)]],
                             ssem[slot], add=True)

        def wait_scatter(slot):
            pltpu.make_async_copy(rows[slot],
                                  acc.at[dcnk[0].at[pl.ds(0, G)]],
                                  ssem[slot]).wait()

        def step(slot, cb, co, qb, qo, issue, wait_prev):
            if issue:
                slot2 = (slot + AHEAD) % NB
                if wait_prev:
                    wait_scatter(slot2)
                issue_gather(slot2, qb, qo)
            wait_gather(slot)
            issue_scatter(slot, cb, co)
            count_deg(cb, co)

        def emit(g):
            if g % CH == 2 and g // CH + 1 < NCHUNK:
                load_chunk((g // CH + 1) % 2,
                           base_e + (g // CH + 1) * CHW)
            q = g + AHEAD
            step(g % NB, (g // CH) % 2, g % CH, (q // CH) % 2, q % CH,
                 q < NFULL, q >= NB)

        load_chunk(0, base_e)
        for q in range(AHEAD):
            issue_gather(q % NB, 0, q)

        zplan = [(s * SPAN + j * ZCH, ZCH) for j in range(SPAN // ZCH)]
        for r0, nr in zplan:
            pltpu.async_copy(zhbm, acc.at[pl.ds(r0, nr)], zsem)

        @pl.when(s == 0)
        def _ztail():
            pltpu.async_copy(zhbm.at[pl.ds(0, REM)],
                             acc.at[pl.ds(REM0, REM)], zsem)

        if count_degrees:
            @pl.loop(0, N // VL)
            def _zd(i):
                mydeg[pl.ds(i * VL, VL)] = jnp.zeros((VL,), jnp.float32)

        for r0, nr in zplan:
            pltpu.make_async_copy(zhbm, acc.at[pl.ds(r0, nr)], zsem).wait()

        @pl.when(s == 0)
        def _ztailw():
            pltpu.make_async_copy(zhbm.at[pl.ds(0, REM)],
                                  acc.at[pl.ds(REM0, REM)], zsem).wait()

        plsc.subcore_barrier()

        for g in range(T0):
            emit(g)

        @pl.loop(T0, TE, step=UNROLL)
        def _edges(t):
            for db in range(UNROLL):
                gg = T0 + db
                if gg % CH == 2:
                    load_chunk((gg // CH + 1) % 2,
                               base_e + ((t + db) // CH + 1) * CHW)
                q = gg + AHEAD
                step(gg % NB, (gg // CH) % 2, gg % CH,
                     (q // CH) % 2, q % CH, True, True)

        for g in range(TE, NFULL):
            emit(g)
        for q in range(NFULL - NB, NFULL):
            wait_scatter(q % NB)

        e0 = base_e + NFULL * G
        tsl = pl.ds(0, TAIL)
        pltpu.sync_copy(src.at[pl.ds(e0, TAIL)], scnk[0].at[tsl])
        pltpu.sync_copy(dst.at[pl.ds(e0, TAIL)], dcnk[0].at[tsl])
        pltpu.sync_copy(table.at[scnk[0].at[tsl]], rows[0].at[tsl])
        pltpu.sync_copy(rows[0].at[tsl], acc.at[dcnk[0].at[tsl]], add=True)
        if count_degrees:
            for j in range(TAIL // VL):
                dv = dcnk[0][pl.ds(j * VL, VL)]
                cnt, last = plsc.scan_count(dv)
                plsc.addupdate_scatter(
                    mydeg, [dv], cnt.astype(jnp.float32), mask=last)

        plsc.subcore_barrier()
        pltpu.sync_copy(acc.at[pl.ds(s * SPAN, SPAN)],
                        out.at[c, pl.ds(s * SPAN, SPAN)])

        @pl.when(s == 0)
        def _otail():
            pltpu.sync_copy(acc.at[pl.ds(REM0, REM)],
                            out.at[c, pl.ds(REM0, REM)])

        if count_degrees:
            pltpu.sync_copy(mydeg, deg_out.at[pl.ds(wid * N, N)])

    return k


def _dot(a, b):
    return jnp.dot(a, b, precision=lax.Precision.HIGHEST,
                   preferred_element_type=jnp.float32)


BN = 2000


def _mean(p_ref, dp_ref, d):
    agg = p_ref[0] + p_ref[1]
    deg = jnp.maximum(jnp.sum(dp_ref[...], axis=1, keepdims=True), 1.0)
    return agg[:, :d] / deg


def _tc_mid_body(x_ref, p0_ref, dp0_ref, ws_ref, wn_ref, b_ref, h1_ref):
    mean = _mean(p0_ref, dp0_ref, D_IN)
    h1_ref[...] = jax.nn.relu(
        _dot(x_ref[...], ws_ref[...]) + _dot(mean, wn_ref[...]) + b_ref[...])


def _tc_out_body(h1_ref, p1_ref, dp1_ref, ws_ref, wn_ref, b_ref, o_ref):
    mean = _mean(p1_ref, dp1_ref, D_H)
    o_ref[...] = (_dot(h1_ref[...], ws_ref[...]) + _dot(mean, wn_ref[...])
                  + b_ref[...])


def _tc_call(body, d_in, d_out):
    return pl.pallas_call(
        body,
        grid=(N // BN,),
        in_specs=[
            pl.BlockSpec((BN, d_in), lambda i: (i, 0)),
            pl.BlockSpec((NC, BN, W), lambda i: (0, i, 0)),
            pl.BlockSpec((BN, NW), lambda i: (i, 0)),
            pl.BlockSpec((d_in, d_out), lambda i: (0, 0)),
            pl.BlockSpec((d_in, d_out), lambda i: (0, 0)),
            pl.BlockSpec((1, d_out), lambda i: (0, 0)),
        ],
        out_specs=pl.BlockSpec((BN, d_out), lambda i: (i, 0)),
        out_shape=jax.ShapeDtypeStruct((N, d_out), jnp.float32),
    )


def kernel(x, edge_index0, edge_index1, W_self0, W_neigh0, b0,
           W_self1, W_neigh1, b1):
    src0, dst0 = edge_index0[0], edge_index0[1]
    src1, dst1 = edge_index1[0], edge_index1[1]
    b0r = b0.reshape(1, D_H)
    b1r = b1.reshape(1, D_OUT)
    sc = _sc_segment_sum(True)

    zs = jnp.zeros((ZCH, W), jnp.float32)
    p0, degf0 = sc(x, src0, dst0, zs)
    dp0 = degf0.reshape(NW, N).T

    h1 = _tc_call(_tc_mid_body, D_IN, D_H)(x, p0, dp0, W_self0, W_neigh0, b0r)

    p1, degf1 = sc(h1, src1, dst1, zs)
    dp1 = degf1.reshape(NW, N).T

    out = _tc_call(_tc_out_body, D_H, D_OUT)(h1, p1, dp1, W_self1, W_neigh1,
                                             b1r)

    return out
